# baseline (speedup 1.0000x reference)
import math

import jax
import jax.numpy as jnp
from jax.experimental import pallas as pl
from jax.experimental.pallas import tpu as pltpu
from jax.experimental.pallas import tpu_sc as plsc

_DIM = 32
_D2 = 64
_K = 1024
_N = 65536
_BETA = 0.25
_TN = 4096
_NT = _N // _TN
_NW = 32


def _vq_body(zt_ref, emb_ref, embm2_ref, xsq_ref, ysq_ref, iota_ref,
             idx_ref, loss_ref, zqr_ref, zqi_ref):
    zt = zt_ref[...]
    emb = emb_ref[...]

    m2dots_t = jax.lax.dot_general(
        embm2_ref[...], zt, (((1,), (0,)), ((), ())),
        preferred_element_type=jnp.float32)
    d_t = xsq_ref[...] + ysq_ref[...] + m2dots_t

    dmin_t = jnp.min(d_t, axis=0, keepdims=True)
    iota_f = iota_ref[...]
    hit = d_t == dmin_t
    idx_f = jnp.min(jnp.where(hit, iota_f, float(_K)), axis=0)
    onehot_t = (iota_f == idx_f[None, :]).astype(jnp.float32)

    zq_t = jax.lax.dot_general(
        emb, onehot_t, (((0,), (0,)), ((), ())),
        preferred_element_type=jnp.float32)

    idx_ref[0, 0, :] = idx_f.astype(jnp.int32)
    loss_ref[0, 0, :] = dmin_t[0, :] * (_BETA / _D2)
    zqr_ref[...] = zq_t[:_DIM, :]
    zqi_ref[...] = zq_t[_DIM:, :]


def _hist_body(idx_hbm, out_hbm, idx_v, ones_v, zero_v, shared):
    c = jax.lax.axis_index("c")
    s = jax.lax.axis_index("s")
    w = c * 16 + s
    pltpu.sync_copy(idx_hbm.at[w], idx_v)
    for t in range(8):
        ones_v[pl.ds(t * 16, 16)] = jnp.full((16,), 1.0, jnp.float32)

    @pl.when(s == 0)
    def _zero_shared():
        for t in range(_K // 16):
            zero_v[pl.ds(t * 16, 16)] = jnp.zeros((16,), jnp.float32)
        pltpu.sync_copy(zero_v, shared)

    plsc.subcore_barrier()
    for j in range(16):
        pltpu.sync_copy(ones_v, shared.at[idx_v.at[j]], add=True)
    plsc.subcore_barrier()

    @pl.when(s == 0)
    def _publish():
        pltpu.sync_copy(shared, out_hbm.at[c])


def _entropy_body(cnt_ref, ent_ref):
    counts = cnt_ref[0:1, :] + cnt_ref[1:2, :]
    p = counts * (1.0 / _N)
    ent = -jnp.sum(p * jnp.log(p + 1e-10), keepdims=True) / math.log(_K)
    ent_ref[...] = ent.reshape(1, 1)


def kernel(z_real, z_imag, embedding):
    z_flat = jnp.concatenate([z_real, z_imag], axis=-1)
    x_sq = jnp.sum(z_flat ** 2, axis=1, keepdims=True)
    y_sq = jnp.sum(embedding ** 2, axis=1)[:, None]
    iota_col = jnp.arange(_K, dtype=jnp.float32)[:, None]
    emb_m2 = -2.0 * embedding

    idx3, loss3, zq_re, zq_im = pl.pallas_call(
        _vq_body,
        grid=(_NT,),
        in_specs=[
            pl.BlockSpec((_D2, _TN), lambda i: (0, i)),
            pl.BlockSpec((_K, _D2), lambda i: (0, 0)),
            pl.BlockSpec((_K, _D2), lambda i: (0, 0)),
            pl.BlockSpec((1, _TN), lambda i: (0, i)),
            pl.BlockSpec((_K, 1), lambda i: (0, 0)),
            pl.BlockSpec((_K, 1), lambda i: (0, 0)),
        ],
        out_specs=[
            pl.BlockSpec((1, 1, _TN), lambda i: (i, 0, 0)),
            pl.BlockSpec((1, 1, _TN), lambda i: (i, 0, 0)),
            pl.BlockSpec((_DIM, _TN), lambda i: (0, i)),
            pl.BlockSpec((_DIM, _TN), lambda i: (0, i)),
        ],
        out_shape=[
            jax.ShapeDtypeStruct((_NT, 1, _TN), jnp.int32),
            jax.ShapeDtypeStruct((_NT, 1, _TN), jnp.float32),
            jax.ShapeDtypeStruct((_DIM, _N), jnp.float32),
            jax.ShapeDtypeStruct((_DIM, _N), jnp.float32),
        ],
    )(z_flat.T, embedding, emb_m2, x_sq.T, y_sq, iota_col)

    indices = idx3.reshape(_N)

    hist = pl.kernel(
        _hist_body,
        out_type=jax.ShapeDtypeStruct((2, _K), jnp.float32),
        mesh=plsc.VectorSubcoreMesh(core_axis_name="c", subcore_axis_name="s"),
        scratch_types=[
            pltpu.VMEM((16, 128), jnp.int32),
            pltpu.VMEM((128,), jnp.float32),
            pltpu.VMEM((_K,), jnp.float32),
            pltpu.VMEM_SHARED((_K,), jnp.float32),
        ],
    )
    counts2 = hist(indices.reshape(_NW, 16, 128))

    ent = pl.pallas_call(
        _entropy_body,
        out_shape=jax.ShapeDtypeStruct((1, 1), jnp.float32),
    )(counts2)

    loss_sample = loss3.reshape(_N)
    z_q_c = jax.lax.complex(zq_re.T, zq_im.T)
    norm_entropy = ent.reshape(())
    return (z_q_c, loss_sample, indices, norm_entropy)


# speedup vs baseline: 1.0070x; 1.0070x over previous
import math

import jax
import jax.numpy as jnp
from jax.experimental import pallas as pl
from jax.experimental.pallas import tpu as pltpu
from jax.experimental.pallas import tpu_sc as plsc

_DIM = 32
_D2 = 64
_K = 1024
_N = 65536
_BETA = 0.25
_TN = 8192
_NT = _N // _TN
_NW = 32


def _vq_body(zt_ref, emb_ref, embm2_ref, xsq_ref, ysq_ref, iota_ref,
             idx_ref, loss_ref, zqr_ref, zqi_ref):
    zt = zt_ref[...]
    emb = emb_ref[...]

    m2dots_t = jax.lax.dot_general(
        embm2_ref[...], zt, (((1,), (0,)), ((), ())),
        preferred_element_type=jnp.float32)
    d_t = xsq_ref[...] + ysq_ref[...] + m2dots_t

    dmin_t = jnp.min(d_t, axis=0, keepdims=True)
    iota_f = iota_ref[...]
    hit = d_t == dmin_t
    idx_f = jnp.min(jnp.where(hit, iota_f, float(_K)), axis=0)
    onehot_t = (iota_f == idx_f[None, :]).astype(jnp.float32)

    zq_t = jax.lax.dot_general(
        emb, onehot_t, (((0,), (0,)), ((), ())),
        preferred_element_type=jnp.float32)

    idx_ref[0, 0, :] = idx_f.astype(jnp.int32)
    loss_ref[0, 0, :] = dmin_t[0, :] * (_BETA / _D2)
    zqr_ref[...] = zq_t[:_DIM, :]
    zqi_ref[...] = zq_t[_DIM:, :]


def _hist_body(idx_hbm, out_hbm, idx_v, ones_v, zero_v, shared):
    c = jax.lax.axis_index("c")
    s = jax.lax.axis_index("s")
    w = c * 16 + s
    pltpu.sync_copy(idx_hbm.at[w], idx_v)
    for t in range(8):
        ones_v[pl.ds(t * 16, 16)] = jnp.full((16,), 1.0, jnp.float32)

    @pl.when(s == 0)
    def _zero_shared():
        for t in range(_K // 16):
            zero_v[pl.ds(t * 16, 16)] = jnp.zeros((16,), jnp.float32)
        pltpu.sync_copy(zero_v, shared)

    plsc.subcore_barrier()
    for j in range(16):
        pltpu.sync_copy(ones_v, shared.at[idx_v.at[j]], add=True)
    plsc.subcore_barrier()

    @pl.when(s == 0)
    def _publish():
        pltpu.sync_copy(shared, out_hbm.at[c])


def _entropy_body(cnt_ref, ent_ref):
    counts = cnt_ref[0:1, :] + cnt_ref[1:2, :]
    p = counts * (1.0 / _N)
    ent = -jnp.sum(p * jnp.log(p + 1e-10), keepdims=True) / math.log(_K)
    ent_ref[...] = ent.reshape(1, 1)


def kernel(z_real, z_imag, embedding):
    z_flat = jnp.concatenate([z_real, z_imag], axis=-1)
    x_sq = jnp.sum(z_flat ** 2, axis=1, keepdims=True)
    y_sq = jnp.sum(embedding ** 2, axis=1)[:, None]
    iota_col = jnp.arange(_K, dtype=jnp.float32)[:, None]
    emb_m2 = -2.0 * embedding

    idx3, loss3, zq_re, zq_im = pl.pallas_call(
        _vq_body,
        grid=(_NT,),
        in_specs=[
            pl.BlockSpec((_D2, _TN), lambda i: (0, i)),
            pl.BlockSpec((_K, _D2), lambda i: (0, 0)),
            pl.BlockSpec((_K, _D2), lambda i: (0, 0)),
            pl.BlockSpec((1, _TN), lambda i: (0, i)),
            pl.BlockSpec((_K, 1), lambda i: (0, 0)),
            pl.BlockSpec((_K, 1), lambda i: (0, 0)),
        ],
        out_specs=[
            pl.BlockSpec((1, 1, _TN), lambda i: (i, 0, 0)),
            pl.BlockSpec((1, 1, _TN), lambda i: (i, 0, 0)),
            pl.BlockSpec((_DIM, _TN), lambda i: (0, i)),
            pl.BlockSpec((_DIM, _TN), lambda i: (0, i)),
        ],
        out_shape=[
            jax.ShapeDtypeStruct((_NT, 1, _TN), jnp.int32),
            jax.ShapeDtypeStruct((_NT, 1, _TN), jnp.float32),
            jax.ShapeDtypeStruct((_DIM, _N), jnp.float32),
            jax.ShapeDtypeStruct((_DIM, _N), jnp.float32),
        ],
    )(z_flat.T, embedding, emb_m2, x_sq.T, y_sq, iota_col)

    indices = idx3.reshape(_N)

    hist = pl.kernel(
        _hist_body,
        out_type=jax.ShapeDtypeStruct((2, _K), jnp.float32),
        mesh=plsc.VectorSubcoreMesh(core_axis_name="c", subcore_axis_name="s"),
        scratch_types=[
            pltpu.VMEM((16, 128), jnp.int32),
            pltpu.VMEM((128,), jnp.float32),
            pltpu.VMEM((_K,), jnp.float32),
            pltpu.VMEM_SHARED((_K,), jnp.float32),
        ],
    )
    counts2 = hist(indices.reshape(_NW, 16, 128))

    ent = pl.pallas_call(
        _entropy_body,
        out_shape=jax.ShapeDtypeStruct((1, 1), jnp.float32),
    )(counts2)

    loss_sample = loss3.reshape(_N)
    z_q_c = jax.lax.complex(zq_re.T, zq_im.T)
    norm_entropy = ent.reshape(())
    return (z_q_c, loss_sample, indices, norm_entropy)
